# baseline (speedup 1.0000x reference)
import functools

import jax
import jax.numpy as jnp
from jax import lax
from jax.experimental import pallas as pl
from jax.experimental.pallas import tpu as pltpu
from jax.experimental.pallas import tpu_sc as plsc

B = 16384
NW = 32
PW = B // NW
CH = 128
NCH = PW // CH
NT = 6
L = 16


def _ncf_body(u2, u2p1, i2, i2p1, uu, ii,
              tgu, tgi, tmu, tmi, wmat,
              out_hbm,
              idx_v, gat_v, w_v, out_v, sem):
    c = lax.axis_index("c")
    s = lax.axis_index("s")
    wid = s * 2 + c

    idx_srcs = (u2, u2p1, i2, i2p1, uu, ii)
    cps = [pltpu.async_copy(src.at[wid], idx_v.at[k], sem)
           for k, src in enumerate(idx_srcs)]
    cps.append(pltpu.async_copy(wmat, w_v, sem))
    for cp in cps:
        cp.wait()

    tabs = (tgu, tgu, tgi, tgi, tmu, tmi)
    gs = []
    for k, tab in enumerate(tabs):
        for j in range(NCH):
            gs.append(pltpu.async_copy(
                tab.at[idx_v.at[k, j]],
                gat_v.at[k, pl.ds(j * CH, CH)],
                sem))
    for g in gs:
        g.wait()

    w = [w_v[r] for r in range(29)]

    for i in range(PW // L):
        dv = pl.ds(i * L, L)
        gu0 = gat_v[0, dv]
        gu1 = gat_v[1, dv]
        gi0 = gat_v[2, dv]
        gi1 = gat_v[3, dv]
        x0 = gat_v[4, dv]
        x1 = gat_v[5, dv]
        g0 = gu0 * gi0
        g1 = gu1 * gi1
        for li in range(4):
            n0 = jnp.maximum(w[4 * li] * x0 + w[4 * li + 1] * x1
                             + w[16 + 2 * li], 0.0)
            n1 = jnp.maximum(w[4 * li + 2] * x0 + w[4 * li + 3] * x1
                             + w[16 + 2 * li + 1], 0.0)
            x0, x1 = n0, n1
        z = w[24] * g0 + w[25] * g1 + w[26] * x0 + w[27] * x1 + w[28]
        out_v[dv] = 1.0 / (1.0 + jnp.exp(-z))

    pltpu.sync_copy(out_v, out_hbm.at[wid])


@jax.jit
def _ncf_sc(u2, u2p1, i2, i2p1, uu, ii, tgu, tgi, tmu, tmi, wmat):
    mesh = plsc.VectorSubcoreMesh(core_axis_name="c", subcore_axis_name="s")
    run = functools.partial(
        pl.kernel,
        out_type=jax.ShapeDtypeStruct((NW, PW), jnp.float32),
        mesh=mesh,
        scratch_types=[
            pltpu.VMEM((NT, NCH, CH), jnp.int32),
            pltpu.VMEM((NT, PW), jnp.float32),
            pltpu.VMEM((29, L), jnp.float32),
            pltpu.VMEM((PW,), jnp.float32),
            pltpu.SemaphoreType.DMA,
        ],
    )(_ncf_body)
    return run(u2, u2p1, i2, i2p1, uu, ii, tgu, tgi, tmu, tmi, wmat)


def kernel(user, item, gmf_user_w, gmf_item_w, mlp_user_w, mlp_item_w,
           fc_W, fc_b, pred_W, pred_b):
    user = user.astype(jnp.int32)
    item = item.astype(jnp.int32)
    shp = (NW, NCH, CH)
    u2 = (user * 2).reshape(shp)
    u2p1 = (user * 2 + 1).reshape(shp)
    i2 = (item * 2).reshape(shp)
    i2p1 = (item * 2 + 1).reshape(shp)
    uu = user.reshape(shp)
    ii = item.reshape(shp)
    w29 = jnp.concatenate([
        fc_W.reshape(-1),
        fc_b.reshape(-1),
        pred_W.reshape(-1),
        pred_b.reshape(-1),
    ])
    wmat = jnp.broadcast_to(w29[:, None], (29, L))
    out = _ncf_sc(u2, u2p1, i2, i2p1, uu, ii,
                  gmf_user_w.reshape(-1), gmf_item_w.reshape(-1),
                  mlp_user_w.reshape(-1), mlp_item_w.reshape(-1), wmat)
    return out.reshape(B, 1)


# speedup vs baseline: 14.0517x; 14.0517x over previous
import functools

import jax
import jax.numpy as jnp
from jax import lax
from jax.experimental import pallas as pl
from jax.experimental.pallas import tpu as pltpu
from jax.experimental.pallas import tpu_sc as plsc

B = 16384
NW = 32
PW = B // NW
CH = 128
NCH = PW // CH
NT = 6
L = 16


def _ncf_body(uu, ii, gu0, gu1, gi0, gi1, mu, mi, wmat,
              out_hbm,
              idx_v, gat_v, w_v, out_v, sem):
    c = lax.axis_index("c")
    s = lax.axis_index("s")
    wid = s * 2 + c

    cps = [pltpu.async_copy(uu.at[wid], idx_v.at[0], sem),
           pltpu.async_copy(ii.at[wid], idx_v.at[1], sem),
           pltpu.async_copy(wmat, w_v, sem)]
    for cp in cps:
        cp.wait()

    tabs = ((gu0, 0), (gu1, 0), (gi0, 1), (gi1, 1), (mu, 0), (mi, 1))
    gs = []
    for t, (tab, which) in enumerate(tabs):
        for j in range(NCH):
            gs.append(pltpu.async_copy(
                tab.at[idx_v.at[which, j]],
                gat_v.at[t, pl.ds(j * CH, CH)],
                sem))
    for g in gs:
        g.wait()

    w = [w_v[r] for r in range(29)]

    for i in range(PW // L):
        dv = pl.ds(i * L, L)
        a0 = gat_v[0, dv]
        a1 = gat_v[1, dv]
        b0 = gat_v[2, dv]
        b1 = gat_v[3, dv]
        x0 = gat_v[4, dv]
        x1 = gat_v[5, dv]
        g0 = a0 * b0
        g1 = a1 * b1
        for li in range(4):
            n0 = jnp.maximum(w[4 * li] * x0 + w[4 * li + 1] * x1
                             + w[16 + 2 * li], 0.0)
            n1 = jnp.maximum(w[4 * li + 2] * x0 + w[4 * li + 3] * x1
                             + w[16 + 2 * li + 1], 0.0)
            x0, x1 = n0, n1
        z = w[24] * g0 + w[25] * g1 + w[26] * x0 + w[27] * x1 + w[28]
        out_v[dv] = 1.0 / (1.0 + jnp.exp(-z))

    pltpu.sync_copy(out_v, out_hbm.at[wid])


@jax.jit
def _ncf_sc(uu, ii, gu0, gu1, gi0, gi1, mu, mi, wmat):
    mesh = plsc.VectorSubcoreMesh(core_axis_name="c", subcore_axis_name="s")
    run = functools.partial(
        pl.kernel,
        out_type=jax.ShapeDtypeStruct((NW, PW), jnp.float32),
        mesh=mesh,
        scratch_types=[
            pltpu.VMEM((2, NCH, CH), jnp.int32),
            pltpu.VMEM((NT, PW), jnp.float32),
            pltpu.VMEM((29, L), jnp.float32),
            pltpu.VMEM((PW,), jnp.float32),
            pltpu.SemaphoreType.DMA,
        ],
    )(_ncf_body)
    return run(uu, ii, gu0, gu1, gi0, gi1, mu, mi, wmat)


def kernel(user, item, gmf_user_w, gmf_item_w, mlp_user_w, mlp_item_w,
           fc_W, fc_b, pred_W, pred_b):
    shp = (NW, NCH, CH)
    uu = user.astype(jnp.int32).reshape(shp)
    ii = item.astype(jnp.int32).reshape(shp)
    gu0 = gmf_user_w[:, 0]
    gu1 = gmf_user_w[:, 1]
    gi0 = gmf_item_w[:, 0]
    gi1 = gmf_item_w[:, 1]
    mu = mlp_user_w[:, 0]
    mi = mlp_item_w[:, 0]
    w29 = jnp.concatenate([
        fc_W.reshape(-1),
        fc_b.reshape(-1),
        pred_W.reshape(-1),
        pred_b.reshape(-1),
    ])
    wmat = jnp.broadcast_to(w29[:, None], (29, L))
    out = _ncf_sc(uu, ii, gu0, gu1, gi0, gi1, mu, mi, wmat)
    return out.reshape(B, 1)


# speedup vs baseline: 51.4451x; 3.6611x over previous
import functools

import jax
import jax.numpy as jnp
from jax import lax
from jax.experimental import pallas as pl
from jax.experimental.pallas import tpu as pltpu
from jax.experimental.pallas import tpu_sc as plsc

B = 16384
NW = 32
PW = B // NW
CH = 128
NCH = PW // CH
NT = 6
L = 16
V = 1000000
BK = 131072
NBK = (V + BK - 1) // BK


def _split_body(gu_ref, gi_ref, mu_ref, mi_ref,
                o0_ref, o1_ref, o2_ref, o3_ref, o4_ref, o5_ref):
    o0_ref[...] = gu_ref[0, :]
    o1_ref[...] = gu_ref[1, :]
    o2_ref[...] = gi_ref[0, :]
    o3_ref[...] = gi_ref[1, :]
    o4_ref[...] = mu_ref[0, :]
    o5_ref[...] = mi_ref[0, :]


def _split_columns(gmf_u, gmf_i, mlp_u, mlp_i):
    row_spec = pl.BlockSpec((2, BK), lambda j: (0, j))
    one_spec = pl.BlockSpec((1, BK), lambda j: (0, j))
    col_spec = pl.BlockSpec((BK,), lambda j: (j,))
    return pl.pallas_call(
        _split_body,
        grid=(NBK,),
        in_specs=[row_spec, row_spec, one_spec, one_spec],
        out_specs=[col_spec] * NT,
        out_shape=[jax.ShapeDtypeStruct((V,), jnp.float32)] * NT,
    )(gmf_u.T, gmf_i.T, mlp_u.T, mlp_i.T)


def _ncf_body(uu, ii, gu0, gu1, gi0, gi1, mu, mi, wmat,
              out_hbm,
              idx_v, gat_v, w_v, out_v, sem):
    c = lax.axis_index("c")
    s = lax.axis_index("s")
    wid = s * 2 + c

    cps = [pltpu.async_copy(uu.at[wid], idx_v.at[0], sem),
           pltpu.async_copy(ii.at[wid], idx_v.at[1], sem),
           pltpu.async_copy(wmat, w_v, sem)]
    for cp in cps:
        cp.wait()

    tabs = ((gu0, 0), (gu1, 0), (gi0, 1), (gi1, 1), (mu, 0), (mi, 1))
    gs = []
    for t, (tab, which) in enumerate(tabs):
        for j in range(NCH):
            gs.append(pltpu.async_copy(
                tab.at[idx_v.at[which, j]],
                gat_v.at[t, pl.ds(j * CH, CH)],
                sem))
    for g in gs:
        g.wait()

    w = [w_v[r] for r in range(29)]

    for i in range(PW // L):
        dv = pl.ds(i * L, L)
        a0 = gat_v[0, dv]
        a1 = gat_v[1, dv]
        b0 = gat_v[2, dv]
        b1 = gat_v[3, dv]
        x0 = gat_v[4, dv]
        x1 = gat_v[5, dv]
        g0 = a0 * b0
        g1 = a1 * b1
        for li in range(4):
            n0 = jnp.maximum(w[4 * li] * x0 + w[4 * li + 1] * x1
                             + w[16 + 2 * li], 0.0)
            n1 = jnp.maximum(w[4 * li + 2] * x0 + w[4 * li + 3] * x1
                             + w[16 + 2 * li + 1], 0.0)
            x0, x1 = n0, n1
        z = w[24] * g0 + w[25] * g1 + w[26] * x0 + w[27] * x1 + w[28]
        out_v[dv] = 1.0 / (1.0 + jnp.exp(-z))

    pltpu.sync_copy(out_v, out_hbm.at[wid])


@jax.jit
def _ncf_sc(uu, ii, gu0, gu1, gi0, gi1, mu, mi, wmat):
    mesh = plsc.VectorSubcoreMesh(core_axis_name="c", subcore_axis_name="s")
    run = functools.partial(
        pl.kernel,
        out_type=jax.ShapeDtypeStruct((NW, PW), jnp.float32),
        mesh=mesh,
        scratch_types=[
            pltpu.VMEM((2, NCH, CH), jnp.int32),
            pltpu.VMEM((NT, PW), jnp.float32),
            pltpu.VMEM((29, L), jnp.float32),
            pltpu.VMEM((PW,), jnp.float32),
            pltpu.SemaphoreType.DMA,
        ],
    )(_ncf_body)
    return run(uu, ii, gu0, gu1, gi0, gi1, mu, mi, wmat)


def kernel(user, item, gmf_user_w, gmf_item_w, mlp_user_w, mlp_item_w,
           fc_W, fc_b, pred_W, pred_b):
    shp = (NW, NCH, CH)
    uu = user.astype(jnp.int32).reshape(shp)
    ii = item.astype(jnp.int32).reshape(shp)
    gu0, gu1, gi0, gi1, mu, mi = _split_columns(
        gmf_user_w, gmf_item_w, mlp_user_w, mlp_item_w)
    w29 = jnp.concatenate([
        fc_W.reshape(-1),
        fc_b.reshape(-1),
        pred_W.reshape(-1),
        pred_b.reshape(-1),
    ])
    wmat = jnp.broadcast_to(w29[:, None], (29, L))
    out = _ncf_sc(uu, ii, gu0, gu1, gi0, gi1, mu, mi, wmat)
    return out.reshape(B, 1)
